# Initial kernel scaffold; baseline (speedup 1.0000x reference)
import jax
import jax.numpy as jnp
from jax.experimental import pallas as pl


def kernel(x, edge_attr, params, edge_index, batch):
    raise NotImplementedError("write your pallas kernel here")


# speedup vs baseline: 8.0254x; 8.0254x over previous
import functools

import jax
import jax.numpy as jnp
from jax import lax
from jax.experimental import pallas as pl
from jax.experimental.pallas import tpu as pltpu
from jax.experimental.pallas import tpu_sc as plsc

F32 = jnp.float32
NC, NS = 2, 16
NW = NC * NS
CH = 80
B_GRAPHS = 256


def _mm(a, w):
    return lax.dot_general(a, w, (((1,), (1,)), ((), ())),
                           preferred_element_type=F32)


def _lrelu(v):
    return jnp.where(v > 0, v, 0.01 * v)


def _elu(v):
    return jnp.where(v > 0, v, jnp.exp(jnp.minimum(v, 0.0)) - 1.0)


def _gru_math(gi, gh, h):
    hdim = h.shape[-1]
    ir, iz, inn = gi[:, :hdim], gi[:, hdim:2 * hdim], gi[:, 2 * hdim:]
    hr, hz, hn = gh[:, :hdim], gh[:, hdim:2 * hdim], gh[:, 2 * hdim:]
    r = jax.nn.sigmoid(ir + hr)
    z = jax.nn.sigmoid(iz + hz)
    n = jnp.tanh(inn + r * hn)
    return (1.0 - z) * n + z * h


def _vecspec(h):
    return pl.BlockSpec((h,), lambda i: (0,))


def _matspec(r, c):
    return pl.BlockSpec((r, c), lambda i: (0, 0))


def _rowspec(bm, c):
    return pl.BlockSpec((bm, c), lambda i: (i, 0))


def _sds(shape):
    return jax.ShapeDtypeStruct(shape, F32)


def _tc_stage_a(x_p, w1, b1, wx, att_r, bm=512):
    n_pad, d = x_p.shape
    h = w1.shape[0]

    def body(x_ref, w1_ref, b1_ref, wx_ref, ar_ref, x1_ref, xw_ref, sr_ref):
        x1 = _lrelu(_mm(x_ref[...], w1_ref[...]) + b1_ref[...])
        x1_ref[...] = x1
        xw_ref[...] = _mm(x1, wx_ref[...])
        sr_ref[...] = jnp.sum(x1 * ar_ref[...], axis=1)

    return pl.pallas_call(
        body,
        grid=(n_pad // bm,),
        in_specs=[_rowspec(bm, d), _matspec(h, d), _vecspec(h),
                  _matspec(h, h), _vecspec(h)],
        out_specs=[_rowspec(bm, h), _rowspec(bm, h),
                   pl.BlockSpec((bm,), lambda i: (i,))],
        out_shape=[_sds((n_pad, h)), _sds((n_pad, h)), _sds((n_pad,))],
    )(x_p, w1, b1, wx, att_r)


def _tc_stage_b(g, ea, we, att_l, bm=512):
    e, h = g.shape
    ed = ea.shape[1]

    def body(g_ref, ea_ref, we_ref, al_ref, t_ref, aj_ref):
        t = _lrelu(g_ref[...] + _mm(ea_ref[...], we_ref[...]))
        t_ref[...] = t
        aj_ref[...] = jnp.sum(t * al_ref[...], axis=1)

    return pl.pallas_call(
        body,
        grid=(e // bm,),
        in_specs=[_rowspec(bm, h), _rowspec(bm, ed), _matspec(h, ed),
                  _vecspec(h)],
        out_specs=[_rowspec(bm, h), pl.BlockSpec((bm,), lambda i: (i,))],
        out_shape=[_sds((e, h)), _sds((e,))],
    )(g, ea, we, att_l)


def _tc_scale_rows(rows, alpha, bm=512):
    k, h = rows.shape

    def body(r_ref, a_ref, o_ref):
        o_ref[...] = r_ref[...] * a_ref[...][:, None]

    return pl.pallas_call(
        body,
        grid=(k // bm,),
        in_specs=[_rowspec(bm, h), pl.BlockSpec((bm,), lambda i: (i,))],
        out_specs=_rowspec(bm, h),
        out_shape=_sds((k, h)),
    )(rows, alpha)


def _tc_stage_e(parts, w2, gb, x1, wih, whh, bih, bhh, gatw, att_s, att_d,
                bm=512):
    n_pad, h = x1.shape

    def body(p_ref, w2_ref, gb_ref, x1_ref, wih_ref, whh_ref, bih_ref,
             bhh_ref, gw_ref, as_ref, ad_ref,
             x2_ref, xs_ref, asv_ref, adv_ref):
        agg = p_ref[0] + p_ref[1]
        hh = _elu(_mm(agg, w2_ref[...]) + gb_ref[...])
        x1v = x1_ref[...]
        gi = _mm(hh, wih_ref[...]) + bih_ref[...]
        gh = _mm(x1v, whh_ref[...]) + bhh_ref[...]
        x2 = jnp.maximum(_gru_math(gi, gh, x1v), 0.0)
        x2_ref[...] = x2
        xs = _mm(x2, gw_ref[...])
        xs_ref[...] = xs
        asv_ref[...] = jnp.sum(xs * as_ref[...], axis=1)
        adv_ref[...] = jnp.sum(xs * ad_ref[...], axis=1)

    return pl.pallas_call(
        body,
        grid=(n_pad // bm,),
        in_specs=[pl.BlockSpec((NC, bm, h), lambda i: (0, i, 0)),
                  _matspec(h, h), _vecspec(h), _rowspec(bm, h),
                  _matspec(3 * h, h), _matspec(3 * h, h),
                  _vecspec(3 * h), _vecspec(3 * h),
                  _matspec(h, h), _vecspec(h), _vecspec(h)],
        out_specs=[_rowspec(bm, h), _rowspec(bm, h),
                   pl.BlockSpec((bm,), lambda i: (i,)),
                   pl.BlockSpec((bm,), lambda i: (i,))],
        out_shape=[_sds((n_pad, h)), _sds((n_pad, h)), _sds((n_pad,)),
                   _sds((n_pad,))],
    )(parts, w2, gb, x1, wih, whh, bih, bhh, gatw, att_s, att_d)


def _tc_stage_f(parts, gbias, x2, wih, whh, bih, bhh, molws, matt_s, bm=512):
    n_pad, h = x2.shape

    def body(p_ref, gb_ref, x2_ref, wih_ref, whh_ref, bih_ref, bhh_ref,
             mw_ref, ms_ref, x3_ref, hs_ref, a3_ref):
        agg = p_ref[0] + p_ref[1]
        hh = _elu(agg + gb_ref[...])
        x2v = x2_ref[...]
        gi = _mm(hh, wih_ref[...]) + bih_ref[...]
        gh = _mm(x2v, whh_ref[...]) + bhh_ref[...]
        x3 = jnp.maximum(_gru_math(gi, gh, x2v), 0.0)
        x3_ref[...] = x3
        hs = _mm(x3, mw_ref[...])
        hs_ref[...] = hs
        a3_ref[...] = jnp.sum(hs * ms_ref[...], axis=1)

    return pl.pallas_call(
        body,
        grid=(n_pad // bm,),
        in_specs=[pl.BlockSpec((NC, bm, h), lambda i: (0, i, 0)),
                  _vecspec(h), _rowspec(bm, h),
                  _matspec(3 * h, h), _matspec(3 * h, h),
                  _vecspec(3 * h), _vecspec(3 * h),
                  _matspec(h, h), _vecspec(h)],
        out_specs=[_rowspec(bm, h), _rowspec(bm, h),
                   pl.BlockSpec((bm,), lambda i: (i,))],
        out_shape=[_sds((n_pad, h)), _sds((n_pad, h)), _sds((n_pad,))],
    )(parts, gbias, x2, wih, whh, bih, bhh, molws, matt_s)


def _tc_stage_g(parts0, molwd, matt_d, b=B_GRAPHS):
    _, tb, h = parts0.shape

    def body(p_ref, mw_ref, md_ref, o_ref, ad_ref):
        o = jnp.maximum(p_ref[0, :b, :] + p_ref[1, :b, :], 0.0)
        o_ref[...] = o
        hd = _mm(o, mw_ref[...])
        ad_ref[...] = jnp.sum(hd * md_ref[...], axis=1)

    return pl.pallas_call(
        body,
        in_specs=[pl.BlockSpec((NC, tb, h), lambda: (0, 0, 0)),
                  pl.BlockSpec((h, h), lambda: (0, 0)),
                  pl.BlockSpec((h,), lambda: (0,))],
        out_specs=[pl.BlockSpec((b, h), lambda: (0, 0)),
                   pl.BlockSpec((b,), lambda: (0,))],
        out_shape=[_sds((b, h)), _sds((b,))],
    )(parts0, molwd, matt_d)


def _tc_stage_h(parts3, mbias, out_prev, wih, whh, bih, bhh, molwd, matt_d,
                l2w, l2b_vec, b=B_GRAPHS):
    _, tb, h = parts3.shape

    def body(p_ref, mb_ref, op_ref, wih_ref, whh_ref, bih_ref, bhh_ref,
             mw_ref, md_ref, lw_ref, lb_ref, o_ref, ad_ref, pr_ref):
        agg = p_ref[0, :b, :] + p_ref[1, :b, :]
        hh = _elu(agg + mb_ref[...])
        op = op_ref[...]
        gi = _mm(hh, wih_ref[...]) + bih_ref[...]
        gh = _mm(op, whh_ref[...]) + bhh_ref[...]
        o = jnp.maximum(_gru_math(gi, gh, op), 0.0)
        o_ref[...] = o
        hd = _mm(o, mw_ref[...])
        ad_ref[...] = jnp.sum(hd * md_ref[...], axis=1)
        pr_ref[...] = jnp.sum(o * lw_ref[...], axis=1) + lb_ref[...]

    return pl.pallas_call(
        body,
        in_specs=[pl.BlockSpec((NC, tb, h), lambda: (0, 0, 0)),
                  pl.BlockSpec((h,), lambda: (0,)),
                  pl.BlockSpec((b, h), lambda: (0, 0)),
                  pl.BlockSpec((3 * h, h), lambda: (0, 0)),
                  pl.BlockSpec((3 * h, h), lambda: (0, 0)),
                  pl.BlockSpec((3 * h,), lambda: (0,)),
                  pl.BlockSpec((3 * h,), lambda: (0,)),
                  pl.BlockSpec((h, h), lambda: (0, 0)),
                  pl.BlockSpec((h,), lambda: (0,)),
                  pl.BlockSpec((h,), lambda: (0,)),
                  pl.BlockSpec((b,), lambda: (0,))],
        out_specs=[pl.BlockSpec((b, h), lambda: (0, 0)),
                   pl.BlockSpec((b,), lambda: (0,)),
                   pl.BlockSpec((b,), lambda: (0,))],
        out_shape=[_sds((b, h)), _sds((b,)), _sds((b,))],
    )(parts3, mbias, out_prev, wih, whh, bih, bhh, molwd, matt_d, l2w,
      l2b_vec)


_SC_MESH = plsc.VectorSubcoreMesh(core_axis_name="c", subcore_axis_name="s",
                                  num_cores=NC, num_subcores=NS)
_SC_PARAMS = pltpu.CompilerParams(needs_layout_passes=False)


@functools.lru_cache(maxsize=None)
def _sc_gather(k_rows, t_rows, h):
    kw = k_rows // NW
    assert k_rows % NW == 0
    gc = min(5, kw // CH)
    gsz = gc * CH
    ng = kw // gsz
    assert kw % gsz == 0

    @functools.partial(
        pl.kernel,
        out_type=_sds((k_rows, h)),
        mesh=_SC_MESH,
        compiler_params=_SC_PARAMS,
        scratch_types=[pltpu.VMEM((gc, CH), jnp.int32),
                       pltpu.VMEM((gsz, h), F32),
                       pltpu.SemaphoreType.DMA,
                       pltpu.SemaphoreType.DMA],
    )
    def k(table_h, idx_h, out_h, idx_v, rows_v, sem_i, sem_g):
        cid = lax.axis_index("c")
        sid = lax.axis_index("s")
        wid = sid * NC + cid
        for g in range(ng):
            b0 = pl.multiple_of(wid * kw + g * gsz, gsz)
            di = [
                pltpu.async_copy(idx_h.at[pl.ds(b0 + j * CH, CH)],
                                 idx_v.at[j], sem_i)
                for j in range(gc)
            ]
            for d_ in di:
                d_.wait()
            descs = [
                pltpu.async_copy(table_h.at[idx_v.at[j]],
                                 rows_v.at[pl.ds(j * CH, CH)], sem_g)
                for j in range(gc)
            ]
            for d_ in descs:
                d_.wait()
            pltpu.sync_copy(rows_v, out_h.at[pl.ds(b0, gsz)])

    return k


@functools.lru_cache(maxsize=None)
def _sc_scatter_add(k_rows, t_rows, h):
    kw = k_rows // NW
    assert k_rows % NW == 0 and kw % CH == 0
    ngc = kw // CH
    ts = t_rows // NS
    assert t_rows % NS == 0

    @functools.partial(
        pl.kernel,
        out_type=_sds((NC, t_rows, h)),
        mesh=_SC_MESH,
        compiler_params=_SC_PARAMS,
        scratch_types=[pltpu.VMEM((2, CH), jnp.int32),
                       pltpu.VMEM((2, CH, h), F32),
                       pltpu.VMEM_SHARED((t_rows, h), F32),
                       pltpu.SemaphoreType.DMA,
                       pltpu.SemaphoreType.DMA,
                       pltpu.SemaphoreType.DMA],
    )
    def k(vals_h, idx_h, out_h, idx_v, rows_v, acc_sh, sem_i, sem_v, sem_s):
        cid = lax.axis_index("c")
        sid = lax.axis_index("s")
        wid = sid * NC + cid

        def zero_body(i, _):
            for l in range(h // 16):
                rows_v[0, i, pl.ds(l * 16, 16)] = jnp.zeros((16,), F32)
            return 0

        lax.fori_loop(0, CH, zero_body, 0)
        off = 0
        while off < ts:
            c = min(CH, ts - off)
            pltpu.sync_copy(rows_v.at[0, pl.ds(0, c)],
                            acc_sh.at[pl.ds(sid * ts + off, c)])
            off += c
        plsc.subcore_barrier()

        base = pl.multiple_of(wid * kw, CH)

        def start(g):
            b = g % 2
            return (pltpu.async_copy(idx_h.at[pl.ds(base + g * CH, CH)],
                                     idx_v.at[b], sem_i),
                    pltpu.async_copy(vals_h.at[pl.ds(base + g * CH, CH)],
                                     rows_v.at[b], sem_v))

        pend = start(0)
        prev_sc = None
        for g in range(ngc):
            b = g % 2
            if g + 1 < ngc:
                if prev_sc is not None:
                    prev_sc.wait()
                    prev_sc = None
                nxt = start(g + 1)
            else:
                nxt = None
            pend[0].wait()
            pend[1].wait()
            if prev_sc is not None:
                prev_sc.wait()
            prev_sc = pltpu.async_copy(rows_v.at[b], acc_sh.at[idx_v.at[b]],
                                       sem_s, add=True)
            pend = nxt
        prev_sc.wait()
        plsc.subcore_barrier()
        pltpu.sync_copy(acc_sh.at[pl.ds(sid * ts, ts)],
                        out_h.at[cid, pl.ds(sid * ts, ts)])

    return k


@functools.lru_cache(maxsize=None)
def _sc_edge_softmax(k_edges, s_size, gather_aj, c_s):
    kw = k_edges // NS
    assert k_edges % NS == 0 and kw % c_s == 0 and c_s % 16 == 0
    nch = kw // c_s
    s_chunk = min(s_size, 2048)
    nred = s_size // s_chunk
    sc16 = s_chunk // NS
    assert s_size % s_chunk == 0 and s_chunk % (NS * 16) == 0

    scratch = [pltpu.VMEM((s_size,), F32),
               pltpu.VMEM((s_size,), F32),
               pltpu.VMEM((s_size,), F32),
               pltpu.VMEM((c_s,), F32),
               pltpu.VMEM((c_s,), jnp.int32),
               pltpu.VMEM((c_s,), jnp.int32),
               pltpu.VMEM((c_s,), F32),
               pltpu.VMEM((NS, sc16), F32),
               pltpu.VMEM((sc16,), F32),
               pltpu.VMEM_SHARED((NS, s_chunk), F32),
               pltpu.VMEM_SHARED((s_size,), F32)]

    @functools.partial(
        pl.kernel,
        out_type=(_sds((k_edges,)), _sds((k_edges,))),
        mesh=_SC_MESH,
        compiler_params=_SC_PARAMS,
        scratch_types=scratch,
    )
    def k(in0_h, in1_h, src_h, dst_h, alpha_h, e_h, tab_v, tab2_v, s_part,
          aj_v, dst_v, src_v, e_v, red_v, sred_v, sall_sh, sfull_sh):
        cid = lax.axis_index("c")
        sid = lax.axis_index("s")

        @pl.when(cid == 0)
        def _():
            if gather_aj:
                pltpu.sync_copy(in0_h, tab_v)
                pltpu.sync_copy(in1_h, tab2_v)
            else:
                pltpu.sync_copy(in1_h, tab_v)

            def zs(i, _):
                s_part[pl.ds(i * 16, 16)] = jnp.zeros((16,), F32)
                return 0

            lax.fori_loop(0, s_size // 16, zs, 0)

            base = sid * kw
            for c in range(nch):
                b0 = pl.multiple_of(base + c * c_s, 16)
                pltpu.sync_copy(dst_h.at[pl.ds(b0, c_s)], dst_v)
                if gather_aj:
                    pltpu.sync_copy(src_h.at[pl.ds(b0, c_s)], src_v)
                else:
                    pltpu.sync_copy(in0_h.at[pl.ds(b0, c_s)], aj_v)

                def step(kk, _):
                    sl = pl.ds(kk * 16, 16)
                    di = dst_v[sl]
                    if gather_aj:
                        aj = plsc.load_gather(tab_v, [src_v[sl]])
                        ai = plsc.load_gather(tab2_v, [di])
                    else:
                        aj = aj_v[sl]
                        ai = plsc.load_gather(tab_v, [di])
                    ev = jnp.exp(_lrelu(aj + ai))
                    e_v[sl] = ev
                    plsc.addupdate_scatter(s_part, [di], ev)
                    return 0

                lax.fori_loop(0, c_s // 16, step, 0)
                pltpu.sync_copy(e_v, e_h.at[pl.ds(b0, c_s)])

            for rr in range(nred):
                pltpu.sync_copy(s_part.at[pl.ds(rr * s_chunk, s_chunk)],
                                sall_sh.at[sid])
                plsc.subcore_barrier()
                pltpu.sync_copy(sall_sh.at[:, pl.ds(sid * sc16, sc16)],
                                red_v)

                def red(g, _):
                    sl = pl.ds(g * 16, 16)
                    acc = red_v[0, sl]
                    for r in range(1, NS):
                        acc = acc + red_v[r, sl]
                    sred_v[sl] = acc
                    return 0

                lax.fori_loop(0, sc16 // 16, red, 0)
                pltpu.sync_copy(
                    sred_v,
                    sfull_sh.at[pl.ds(rr * s_chunk + sid * sc16, sc16)])
                plsc.subcore_barrier()
            pltpu.sync_copy(sfull_sh, s_part)

            for c in range(nch):
                b0 = pl.multiple_of(base + c * c_s, 16)
                pltpu.sync_copy(dst_h.at[pl.ds(b0, c_s)], dst_v)
                pltpu.sync_copy(e_h.at[pl.ds(b0, c_s)], e_v)

                def step2(kk, _):
                    sl = pl.ds(kk * 16, 16)
                    sv = plsc.load_gather(s_part, [dst_v[sl]])
                    aj_v[sl] = e_v[sl] / (sv + 1e-16)
                    return 0

                lax.fori_loop(0, c_s // 16, step2, 0)
                pltpu.sync_copy(aj_v, alpha_h.at[pl.ds(b0, c_s)])

    return k


def kernel(x, edge_attr, params, edge_index, batch):
    p = params
    n, d = x.shape
    e = edge_index.shape[1]
    h = p["lin1_W"].shape[0]
    b = B_GRAPHS
    n_pad = -(-n // 512) * 512

    src = edge_index[0]
    dst = edge_index[1]

    x_p = jnp.pad(x, ((0, n_pad - n), (0, 0)))
    batch_p = jnp.concatenate(
        [batch, jnp.full((n_pad - n,), b, jnp.int32)])

    wx = p["g_lin1_W"][:, :h]
    we = p["g_lin1_W"][:, h:]

    x1, xw, s_r = _tc_stage_a(x_p, p["lin1_W"], p["lin1_b"], wx, p["g_att_r"])

    g_rows = _sc_gather(e, n_pad, h)(xw, src)
    t, aj = _tc_stage_b(g_rows, edge_attr, we, p["g_att_l"])
    alpha1, _ = _sc_edge_softmax(e, n_pad, False, 2000)(aj, s_r, src, dst)
    t_a = _tc_scale_rows(t, alpha1)
    parts1 = _sc_scatter_add(e, n_pad, h)(t_a, dst)

    gru0 = p["gru0"]
    x2, xs, a_s, a_d = _tc_stage_e(
        parts1, p["g_lin2_W"], p["g_bias"], x1,
        gru0["Wih"], gru0["Whh"], gru0["bih"], gru0["bhh"],
        p["gat_W"], p["gat_att_src"], p["gat_att_dst"])

    alpha2, _ = _sc_edge_softmax(e, n_pad, True, 2000)(a_s, a_d, src, dst)
    g2 = _sc_gather(e, n_pad, h)(xs, src)
    g2_a = _tc_scale_rows(g2, alpha2)
    parts2 = _sc_scatter_add(e, n_pad, h)(g2_a, dst)

    gru1 = p["gru1"]
    x3, hs, a3 = _tc_stage_f(
        parts2, p["gat_bias"], x2,
        gru1["Wih"], gru1["Whh"], gru1["bih"], gru1["bhh"],
        p["mol_W_src"], p["mol_att_src"])

    tb = 384
    parts0 = _sc_scatter_add(n_pad, tb, h)(x3, batch_p)
    out_g, ad3 = _tc_stage_g(parts0, p["mol_W_dst"], p["mol_att_dst"])

    mgru = p["mol_gru"]
    l2w = p["lin2_W"][0]
    l2b = jnp.broadcast_to(p["lin2_b"], (b,))
    s_mol = 2048
    pred = None
    for _ in range(2):
        ad3_p = jnp.pad(ad3, (0, s_mol - b))
        alpha3, _ = _sc_edge_softmax(n_pad, s_mol, False, n_pad // NS)(
            a3, ad3_p, batch_p, batch_p)
        hs_a = _tc_scale_rows(hs, alpha3)
        parts3 = _sc_scatter_add(n_pad, tb, h)(hs_a, batch_p)
        out_g, ad3, pred = _tc_stage_h(
            parts3, p["mol_bias"], out_g,
            mgru["Wih"], mgru["Whh"], mgru["bih"], mgru["bhh"],
            p["mol_W_dst"], p["mol_att_dst"], l2w, l2b)

    return pred, out_g


# speedup vs baseline: 10.2552x; 1.2779x over previous
import functools

import jax
import jax.numpy as jnp
from jax import lax
from jax.experimental import pallas as pl
from jax.experimental.pallas import tpu as pltpu
from jax.experimental.pallas import tpu_sc as plsc

F32 = jnp.float32
NC, NS = 2, 16
NW = NC * NS
CH = 80
B_GRAPHS = 256
EPS = 1e-16


def _mm(a, w):
    return lax.dot_general(a, w, (((1,), (1,)), ((), ())),
                           preferred_element_type=F32)


def _lrelu(v):
    return jnp.where(v > 0, v, 0.01 * v)


def _elu(v):
    return jnp.where(v > 0, v, jnp.exp(jnp.minimum(v, 0.0)) - 1.0)


def _gru_math(gi, gh, h):
    hdim = h.shape[-1]
    ir, iz, inn = gi[:, :hdim], gi[:, hdim:2 * hdim], gi[:, 2 * hdim:]
    hr, hz, hn = gh[:, :hdim], gh[:, hdim:2 * hdim], gh[:, 2 * hdim:]
    r = jax.nn.sigmoid(ir + hr)
    z = jax.nn.sigmoid(iz + hz)
    n = jnp.tanh(inn + r * hn)
    return (1.0 - z) * n + z * h


def _vecspec(h):
    return pl.BlockSpec((h,), lambda i: (0,))


def _matspec(r, c):
    return pl.BlockSpec((r, c), lambda i: (0, 0))


def _rowspec(bm, c):
    return pl.BlockSpec((bm, c), lambda i: (i, 0))


def _blkspec(bm):
    return pl.BlockSpec((bm,), lambda i: (i,))


def _sds(shape):
    return jax.ShapeDtypeStruct(shape, F32)


def _tc_stage_a(x_p, w1, b1, wx, att_r, bm=512):
    n_pad, d = x_p.shape
    h = w1.shape[0]

    def body(x_ref, w1_ref, b1_ref, wx_ref, ar_ref, x1_ref, xw_ref, sr_ref):
        x1 = _lrelu(_mm(x_ref[...], w1_ref[...]) + b1_ref[...])
        x1_ref[...] = x1
        xw_ref[...] = _mm(x1, wx_ref[...])
        sr_ref[...] = jnp.sum(x1 * ar_ref[...], axis=1)

    return pl.pallas_call(
        body,
        grid=(n_pad // bm,),
        in_specs=[_rowspec(bm, d), _matspec(h, d), _vecspec(h), _matspec(h, h),
                  _vecspec(h)],
        out_specs=[_rowspec(bm, h), _rowspec(bm, h), _blkspec(bm)],
        out_shape=[_sds((n_pad, h)), _sds((n_pad, h)), _sds((n_pad,))],
    )(x_p, w1, b1, wx, att_r)


def _tc_stage_b(g, ea, ai_e, we, att_l, bm=512):
    e_rows, h = g.shape
    ed = ea.shape[1]

    def body(g_ref, ea_ref, ai_ref, we_ref, al_ref, te_ref, e_ref):
        t = _lrelu(g_ref[...] + _mm(ea_ref[...], we_ref[...]))
        aj = jnp.sum(t * al_ref[...], axis=1)
        ev = jnp.exp(_lrelu(aj + ai_ref[...]))
        te_ref[...] = t * ev[:, None]
        e_ref[...] = ev

    return pl.pallas_call(
        body,
        grid=(e_rows // bm,),
        in_specs=[_rowspec(bm, h), _rowspec(bm, ed), _blkspec(bm),
                  _matspec(h, ed), _vecspec(h)],
        out_specs=[_rowspec(bm, h), _blkspec(bm)],
        out_shape=[_sds((e_rows, h)), _sds((e_rows,))],
    )(g, ea, ai_e, we, att_l)


def _tc_stage_b2(g2, as_e, ad_e, bm=512):
    e_rows, h = g2.shape

    def body(g_ref, as_ref, ad_ref, ge_ref, e_ref):
        ev = jnp.exp(_lrelu(as_ref[...] + ad_ref[...]))
        ge_ref[...] = g_ref[...] * ev[:, None]
        e_ref[...] = ev

    return pl.pallas_call(
        body,
        grid=(e_rows // bm,),
        in_specs=[_rowspec(bm, h), _blkspec(bm), _blkspec(bm)],
        out_specs=[_rowspec(bm, h), _blkspec(bm)],
        out_shape=[_sds((e_rows, h)), _sds((e_rows,))],
    )(g2, as_e, ad_e)


def _tc_mol_pre(hs, a3, batch_p, ad3, b=B_GRAPHS, bm=512):
    n_pad, h = hs.shape

    def body(hs_ref, a3_ref, bt_ref, ad_ref, hse_ref, e_ref):
        bt = bt_ref[...]
        eq = bt[:, None] == lax.broadcasted_iota(jnp.int32, (bm, b), 1)
        adg = jnp.sum(jnp.where(eq, ad_ref[...][None, :], 0.0), axis=1)
        ev = jnp.exp(_lrelu(a3_ref[...] + adg))
        hse_ref[...] = hs_ref[...] * ev[:, None]
        e_ref[...] = ev

    return pl.pallas_call(
        body,
        grid=(n_pad // bm,),
        in_specs=[_rowspec(bm, h), _blkspec(bm), _blkspec(bm), _vecspec(b)],
        out_specs=[_rowspec(bm, h), _blkspec(bm)],
        out_shape=[_sds((n_pad, h)), _sds((n_pad,))],
    )(hs, a3, batch_p, ad3)


def _tc_stage_e(parts, sp, w2, gb, x1, wih, whh, bih, bhh, gatw, att_s, att_d,
                bm=512):
    n_pad, h = x1.shape

    def body(p_ref, sp_ref, w2_ref, gb_ref, x1_ref, wih_ref, whh_ref, bih_ref,
             bhh_ref, gw_ref, as_ref, ad_ref,
             x2_ref, xs_ref, asv_ref, adv_ref):
        s = jnp.sum(sp_ref[...], axis=0) + EPS
        agg = (p_ref[0] + p_ref[1]) / s[:, None]
        hh = _elu(_mm(agg, w2_ref[...]) + gb_ref[...])
        x1v = x1_ref[...]
        gi = _mm(hh, wih_ref[...]) + bih_ref[...]
        gh = _mm(x1v, whh_ref[...]) + bhh_ref[...]
        x2 = jnp.maximum(_gru_math(gi, gh, x1v), 0.0)
        x2_ref[...] = x2
        xs = _mm(x2, gw_ref[...])
        xs_ref[...] = xs
        asv_ref[...] = jnp.sum(xs * as_ref[...], axis=1)
        adv_ref[...] = jnp.sum(xs * ad_ref[...], axis=1)

    return pl.pallas_call(
        body,
        grid=(n_pad // bm,),
        in_specs=[pl.BlockSpec((NC, bm, h), lambda i: (0, i, 0)),
                  pl.BlockSpec((NW, bm), lambda i: (0, i)),
                  _matspec(h, h), _vecspec(h), _rowspec(bm, h),
                  _matspec(3 * h, h), _matspec(3 * h, h),
                  _vecspec(3 * h), _vecspec(3 * h),
                  _matspec(h, h), _vecspec(h), _vecspec(h)],
        out_specs=[_rowspec(bm, h), _rowspec(bm, h), _blkspec(bm),
                   _blkspec(bm)],
        out_shape=[_sds((n_pad, h)), _sds((n_pad, h)), _sds((n_pad,)),
                   _sds((n_pad,))],
    )(parts, sp, w2, gb, x1, wih, whh, bih, bhh, gatw, att_s, att_d)


def _tc_stage_f(parts, sp, gbias, x2, wih, whh, bih, bhh, molws, matt_s,
                bm=512):
    n_pad, h = x2.shape

    def body(p_ref, sp_ref, gb_ref, x2_ref, wih_ref, whh_ref, bih_ref,
             bhh_ref, mw_ref, ms_ref, x3_ref, hs_ref, a3_ref):
        s = jnp.sum(sp_ref[...], axis=0) + EPS
        agg = (p_ref[0] + p_ref[1]) / s[:, None]
        hh = _elu(agg + gb_ref[...])
        x2v = x2_ref[...]
        gi = _mm(hh, wih_ref[...]) + bih_ref[...]
        gh = _mm(x2v, whh_ref[...]) + bhh_ref[...]
        x3 = jnp.maximum(_gru_math(gi, gh, x2v), 0.0)
        x3_ref[...] = x3
        hs = _mm(x3, mw_ref[...])
        hs_ref[...] = hs
        a3_ref[...] = jnp.sum(hs * ms_ref[...], axis=1)

    return pl.pallas_call(
        body,
        grid=(n_pad // bm,),
        in_specs=[pl.BlockSpec((NC, bm, h), lambda i: (0, i, 0)),
                  pl.BlockSpec((NW, bm), lambda i: (0, i)),
                  _vecspec(h), _rowspec(bm, h),
                  _matspec(3 * h, h), _matspec(3 * h, h),
                  _vecspec(3 * h), _vecspec(3 * h),
                  _matspec(h, h), _vecspec(h)],
        out_specs=[_rowspec(bm, h), _rowspec(bm, h), _blkspec(bm)],
        out_shape=[_sds((n_pad, h)), _sds((n_pad, h)), _sds((n_pad,))],
    )(parts, sp, gbias, x2, wih, whh, bih, bhh, molws, matt_s)


def _tc_stage_g(parts0, molwd, matt_d, b=B_GRAPHS):
    _, tb, h = parts0.shape

    def body(p_ref, mw_ref, md_ref, o_ref, ad_ref):
        o = jnp.maximum(p_ref[0, :b, :] + p_ref[1, :b, :], 0.0)
        o_ref[...] = o
        hd = _mm(o, mw_ref[...])
        ad_ref[...] = jnp.sum(hd * md_ref[...], axis=1)

    return pl.pallas_call(
        body,
        in_specs=[pl.BlockSpec((NC, tb, h), lambda: (0, 0, 0)),
                  pl.BlockSpec((h, h), lambda: (0, 0)),
                  pl.BlockSpec((h,), lambda: (0,))],
        out_specs=[pl.BlockSpec((b, h), lambda: (0, 0)),
                   pl.BlockSpec((b,), lambda: (0,))],
        out_shape=[_sds((b, h)), _sds((b,))],
    )(parts0, molwd, matt_d)


def _tc_stage_h(parts3, sp3, mbias, out_prev, wih, whh, bih, bhh, molwd,
                matt_d, l2w, l2b_vec, b=B_GRAPHS):
    _, tb, h = parts3.shape

    def body(p_ref, sp_ref, mb_ref, op_ref, wih_ref, whh_ref, bih_ref,
             bhh_ref, mw_ref, md_ref, lw_ref, lb_ref, o_ref, ad_ref, pr_ref):
        s = jnp.sum(sp_ref[...], axis=0)[:b] + EPS
        agg = (p_ref[0, :b, :] + p_ref[1, :b, :]) / s[:, None]
        hh = _elu(agg + mb_ref[...])
        op = op_ref[...]
        gi = _mm(hh, wih_ref[...]) + bih_ref[...]
        gh = _mm(op, whh_ref[...]) + bhh_ref[...]
        o = jnp.maximum(_gru_math(gi, gh, op), 0.0)
        o_ref[...] = o
        hd = _mm(o, mw_ref[...])
        ad_ref[...] = jnp.sum(hd * md_ref[...], axis=1)
        pr_ref[...] = jnp.sum(o * lw_ref[...], axis=1) + lb_ref[...]

    return pl.pallas_call(
        body,
        in_specs=[pl.BlockSpec((NC, tb, h), lambda: (0, 0, 0)),
                  pl.BlockSpec((NW, tb), lambda: (0, 0)),
                  pl.BlockSpec((h,), lambda: (0,)),
                  pl.BlockSpec((b, h), lambda: (0, 0)),
                  pl.BlockSpec((3 * h, h), lambda: (0, 0)),
                  pl.BlockSpec((3 * h, h), lambda: (0, 0)),
                  pl.BlockSpec((3 * h,), lambda: (0,)),
                  pl.BlockSpec((3 * h,), lambda: (0,)),
                  pl.BlockSpec((h, h), lambda: (0, 0)),
                  pl.BlockSpec((h,), lambda: (0,)),
                  pl.BlockSpec((h,), lambda: (0,)),
                  pl.BlockSpec((b,), lambda: (0,))],
        out_specs=[pl.BlockSpec((b, h), lambda: (0, 0)),
                   pl.BlockSpec((b,), lambda: (0,)),
                   pl.BlockSpec((b,), lambda: (0,))],
        out_shape=[_sds((b, h)), _sds((b,)), _sds((b,))],
    )(parts3, sp3, mbias, out_prev, wih, whh, bih, bhh, molwd, matt_d, l2w,
      l2b_vec)


_SC_MESH = plsc.VectorSubcoreMesh(core_axis_name="c", subcore_axis_name="s",
                                  num_cores=NC, num_subcores=NS)
_SC_PARAMS = pltpu.CompilerParams(needs_layout_passes=False)


@functools.lru_cache(maxsize=None)
def _sc_gather(k_rows, t_rows, h, scalar_srcs):
    kw = k_rows // NW
    assert k_rows % NW == 0
    gc = min(5, kw // CH)
    gsz = gc * CH
    ng = kw // gsz
    assert kw % gsz == 0
    nsc = len(scalar_srcs)
    need_b = "B" in scalar_srcs

    scratch = ([pltpu.VMEM((gc, CH), jnp.int32),
                pltpu.VMEM((gc, CH), jnp.int32),
                pltpu.VMEM((gsz, h), F32)]
               + [pltpu.VMEM((t_rows,), F32) for _ in range(nsc)]
               + [pltpu.VMEM((gsz,), F32) for _ in range(nsc)]
               + [pltpu.SemaphoreType.DMA, pltpu.SemaphoreType.DMA])

    @functools.partial(
        pl.kernel,
        out_type=tuple([_sds((k_rows, h))] + [_sds((k_rows,))] * nsc),
        mesh=_SC_MESH,
        compiler_params=_SC_PARAMS,
        scratch_types=scratch,
    )
    def k(table_h, *refs):
        vecs_h = refs[:nsc]
        idxa_h, idxb_h, rows_out = refs[nsc], refs[nsc + 1], refs[nsc + 2]
        souts = refs[nsc + 3:nsc + 3 + nsc]
        sc = refs[nsc + 3 + nsc:]
        idxa_v, idxb_v, rows_v = sc[0], sc[1], sc[2]
        tabs = sc[3:3 + nsc]
        sbufs = sc[3 + nsc:3 + 2 * nsc]
        sem_i, sem_g = sc[3 + 2 * nsc], sc[4 + 2 * nsc]

        cid = lax.axis_index("c")
        sid = lax.axis_index("s")
        wid = sid * NC + cid
        for t_ in range(nsc):
            pltpu.sync_copy(vecs_h[t_], tabs[t_])
        for g in range(ng):
            b0 = pl.multiple_of(wid * kw + g * gsz, gsz)
            di = [pltpu.async_copy(idxa_h.at[pl.ds(b0 + j * CH, CH)],
                                   idxa_v.at[j], sem_i) for j in range(gc)]
            if need_b:
                di += [pltpu.async_copy(idxb_h.at[pl.ds(b0 + j * CH, CH)],
                                        idxb_v.at[j], sem_i)
                       for j in range(gc)]
            for d_ in di:
                d_.wait()
            descs = [pltpu.async_copy(table_h.at[idxa_v.at[j]],
                                      rows_v.at[pl.ds(j * CH, CH)], sem_g)
                     for j in range(gc)]
            for t_ in range(nsc):
                iv = idxa_v if scalar_srcs[t_] == "A" else idxb_v

                def sg(i, _, t_=t_, iv=iv):
                    jj = i // (CH // 16)
                    kk = i % (CH // 16)
                    idx = iv[jj, pl.ds(kk * 16, 16)]
                    sbufs[t_][pl.ds(i * 16, 16)] = \
                        plsc.load_gather(tabs[t_], [idx])
                    return 0

                lax.fori_loop(0, gsz // 16, sg, 0)
            for d_ in descs:
                d_.wait()
            pltpu.sync_copy(rows_v, rows_out.at[pl.ds(b0, gsz)])
            for t_ in range(nsc):
                pltpu.sync_copy(sbufs[t_], souts[t_].at[pl.ds(b0, gsz)])

    return k


@functools.lru_cache(maxsize=None)
def _sc_scatter_add(k_rows, t_rows, h, with_scalar):
    kw = k_rows // NW
    assert k_rows % NW == 0 and kw % CH == 0
    ngc = kw // CH
    ts = t_rows // NS
    assert t_rows % NS == 0

    scratch = [pltpu.VMEM((2, CH), jnp.int32),
               pltpu.VMEM((2, CH, h), F32),
               pltpu.VMEM((2, CH), F32),
               pltpu.VMEM((t_rows,), F32),
               pltpu.VMEM_SHARED((t_rows, h), F32),
               pltpu.SemaphoreType.DMA,
               pltpu.SemaphoreType.DMA,
               pltpu.SemaphoreType.DMA,
               pltpu.SemaphoreType.DMA]

    @functools.partial(
        pl.kernel,
        out_type=(_sds((NC, t_rows, h)), _sds((NW, t_rows))),
        mesh=_SC_MESH,
        compiler_params=_SC_PARAMS,
        scratch_types=scratch,
    )
    def k(vals_h, svals_h, idx_h, out_h, sout_h, idx_v, rows_v, ev_v, sp_v,
          acc_sh, sem_i, sem_v, sem_e, sem_s):
        cid = lax.axis_index("c")
        sid = lax.axis_index("s")
        wid = sid * NC + cid

        def zero_body(i, _):
            for l in range(h // 16):
                rows_v[0, i, pl.ds(l * 16, 16)] = jnp.zeros((16,), F32)
            return 0

        lax.fori_loop(0, CH, zero_body, 0)

        def zs(i, _):
            sp_v[pl.ds(i * 16, 16)] = jnp.zeros((16,), F32)
            return 0

        lax.fori_loop(0, t_rows // 16, zs, 0)
        off = 0
        while off < ts:
            c = min(CH, ts - off)
            pltpu.sync_copy(rows_v.at[0, pl.ds(0, c)],
                            acc_sh.at[pl.ds(sid * ts + off, c)])
            off += c
        plsc.subcore_barrier()

        base = pl.multiple_of(wid * kw, CH)

        def start(g):
            b = g % 2
            d = [pltpu.async_copy(idx_h.at[pl.ds(base + g * CH, CH)],
                                  idx_v.at[b], sem_i),
                 pltpu.async_copy(vals_h.at[pl.ds(base + g * CH, CH)],
                                  rows_v.at[b], sem_v)]
            if with_scalar:
                d.append(pltpu.async_copy(svals_h.at[pl.ds(base + g * CH, CH)],
                                          ev_v.at[b], sem_e))
            return d

        pend = start(0)
        prev_sc = None
        for g in range(ngc):
            b = g % 2
            if g + 1 < ngc:
                if prev_sc is not None:
                    prev_sc.wait()
                    prev_sc = None
                nxt = start(g + 1)
            else:
                nxt = None
            for d_ in pend:
                d_.wait()
            if prev_sc is not None:
                prev_sc.wait()
            prev_sc = pltpu.async_copy(rows_v.at[b], acc_sh.at[idx_v.at[b]],
                                       sem_s, add=True)
            if with_scalar:
                def ss(i, _, b=b):
                    sl = pl.ds(i * 16, 16)
                    plsc.addupdate_scatter(sp_v, [idx_v[b, sl]], ev_v[b, sl])
                    return 0

                lax.fori_loop(0, CH // 16, ss, 0)
            pend = nxt
        prev_sc.wait()
        plsc.subcore_barrier()
        pltpu.sync_copy(acc_sh.at[pl.ds(sid * ts, ts)],
                        out_h.at[cid, pl.ds(sid * ts, ts)])
        pltpu.sync_copy(sp_v, sout_h.at[wid])

    return k


def kernel(x, edge_attr, params, edge_index, batch):
    p = params
    n, d = x.shape
    e = edge_index.shape[1]
    h = p["lin1_W"].shape[0]
    b = B_GRAPHS
    n_pad = -(-n // 512) * 512

    src = edge_index[0]
    dst = edge_index[1]

    x_p = jnp.pad(x, ((0, n_pad - n), (0, 0)))
    batch_p = jnp.concatenate(
        [batch, jnp.full((n_pad - n,), b, jnp.int32)])
    zeros_k = jnp.zeros((e,), F32)
    zeros_n = jnp.zeros((n_pad,), F32)

    wx = p["g_lin1_W"][:, :h]
    we = p["g_lin1_W"][:, h:]

    x1, xw, s_r = _tc_stage_a(x_p, p["lin1_W"], p["lin1_b"], wx, p["g_att_r"])

    g_rows, ai_e = _sc_gather(e, n_pad, h, ("B",))(xw, s_r, src, dst)
    te, e1 = _tc_stage_b(g_rows, edge_attr, ai_e, we, p["g_att_l"])
    parts1, sp1 = _sc_scatter_add(e, n_pad, h, True)(te, e1, dst)

    gru0 = p["gru0"]
    x2, xs, a_s, a_d = _tc_stage_e(
        parts1, sp1, p["g_lin2_W"], p["g_bias"], x1,
        gru0["Wih"], gru0["Whh"], gru0["bih"], gru0["bhh"],
        p["gat_W"], p["gat_att_src"], p["gat_att_dst"])

    g2, as_e, ad_e = _sc_gather(e, n_pad, h, ("A", "B"))(xs, a_s, a_d, src,
                                                         dst)
    g2e, e2 = _tc_stage_b2(g2, as_e, ad_e)
    parts2, sp2 = _sc_scatter_add(e, n_pad, h, True)(g2e, e2, dst)

    gru1 = p["gru1"]
    x3, hs, a3 = _tc_stage_f(
        parts2, sp2, p["gat_bias"], x2,
        gru1["Wih"], gru1["Whh"], gru1["bih"], gru1["bhh"],
        p["mol_W_src"], p["mol_att_src"])

    tb = 384
    parts0, _ = _sc_scatter_add(n_pad, tb, h, False)(x3, zeros_n, batch_p)
    out_g, ad3 = _tc_stage_g(parts0, p["mol_W_dst"], p["mol_att_dst"])

    mgru = p["mol_gru"]
    l2w = p["lin2_W"][0]
    l2b = jnp.broadcast_to(p["lin2_b"], (b,))
    pred = None
    for _ in range(2):
        hse, e3 = _tc_mol_pre(hs, a3, batch_p, ad3)
        parts3, sp3 = _sc_scatter_add(n_pad, tb, h, True)(hse, e3, batch_p)
        out_g, ad3, pred = _tc_stage_h(
            parts3, sp3, p["mol_bias"], out_g,
            mgru["Wih"], mgru["Whh"], mgru["bih"], mgru["bhh"],
            p["mol_W_dst"], p["mol_att_dst"], l2w, l2b)

    return pred, out_g
